# baseline (speedup 1.0000x reference)
import functools

import jax
import jax.numpy as jnp
from jax.experimental import pallas as pl
from jax.experimental.pallas import tpu as pltpu

_TAU = 0.9
_TOP_P = 0.8
_B = 64


def _router_body(p_ref, w_ref, cb_ref, fcw_ref, fcb_ref, o_ref, pooled_ref):
    w = w_ref[...]
    cb = cb_ref[...]

    def tok(t, _):
        pt = p_ref[t]
        x = jax.lax.dot_general(
            pt, w, (((0,), (1,)), ((), ())),
            preferred_element_type=jnp.float32)
        x = jnp.maximum(x + cb, 0.0)
        pooled_ref[pl.ds(t, 1), :] = jnp.sum(x, axis=0, keepdims=True) * (1.0 / 64.0)
        return 0

    jax.lax.fori_loop(0, _B, tok, 0)

    pooled = pooled_ref[...]
    logits = jax.lax.dot_general(
        pooled, fcw_ref[...], (((1,), (0,)), ((), ())),
        preferred_element_type=jnp.float32) + fcb_ref[...]

    li = logits * (1.0 / _TAU)
    li = li - jnp.max(li, axis=-1, keepdims=True)
    e = jnp.exp(li)
    probs = e / jnp.sum(e, axis=-1, keepdims=True)

    pi = probs[:, :, None]
    pj = probs[:, None, :]
    ii = jax.lax.broadcasted_iota(jnp.int32, (_B, 16, 16), 1)
    jj = jax.lax.broadcasted_iota(jnp.int32, (_B, 16, 16), 2)
    g = (pj > pi) | ((pj == pi) & (jj <= ii))
    s = jnp.sum(jnp.where(g, jnp.broadcast_to(pj, (_B, 16, 16)), 0.0), axis=2)
    cnt = jnp.sum(g.astype(jnp.int32), axis=2)
    keep = (s <= _TOP_P) | (cnt < 2)
    masked = jnp.where(keep, probs, 0.0)
    denom = jnp.clip(jnp.sum(masked, axis=-1, keepdims=True), 1e-10, None)
    o_ref[...] = masked / denom


def kernel(patch, conv_w, conv_b, fc_w, fc_b, layer_idx, threshold):
    del layer_idx, threshold
    n_tok = patch.shape[0]
    hw = patch.shape[2] * patch.shape[3]
    p3 = patch.reshape(n_tok, patch.shape[1], hw)

    grid = (n_tok // _B,)
    out = pl.pallas_call(
        _router_body,
        grid=grid,
        in_specs=[
            pl.BlockSpec((_B, p3.shape[1], hw), lambda i: (i, 0, 0)),
            pl.BlockSpec((128, 196), lambda i: (0, 0)),
            pl.BlockSpec((1, 128), lambda i: (0, 0)),
            pl.BlockSpec((128, 16), lambda i: (0, 0)),
            pl.BlockSpec((1, 16), lambda i: (0, 0)),
        ],
        out_specs=pl.BlockSpec((_B, 16), lambda i: (i, 0)),
        out_shape=jax.ShapeDtypeStruct((n_tok, 16), jnp.float32),
        scratch_shapes=[pltpu.VMEM((_B, 128), jnp.float32)],
    )(p3, conv_w, conv_b.reshape(1, 128), fc_w.T, fc_b.reshape(1, 16))
    return out


# speedup vs baseline: 2.4990x; 2.4990x over previous
import functools

import jax
import jax.numpy as jnp
from jax.experimental import pallas as pl
from jax.experimental.pallas import tpu as pltpu

_TAU = 0.9
_TOP_P = 0.8
_B = 64


def _router_body(p_ref, w_ref, cb_ref, fcw_ref, fcb_ref, o_ref, pooled_ref):
    w = w_ref[...]
    cb = cb_ref[...]

    for grp in range(_B // 8):
        x8 = jnp.concatenate([p_ref[grp * 8 + i] for i in range(8)], axis=1)
        h = jax.lax.dot_general(
            x8, w, (((0,), (1,)), ((), ())),
            preferred_element_type=jnp.float32)
        h = jnp.maximum(h + cb, 0.0)
        hp = jnp.sum(h.reshape(8, 64, 128), axis=1) * (1.0 / 64.0)
        pooled_ref[pl.ds(grp * 8, 8), :] = hp

    pooled = pooled_ref[...]
    logits = jax.lax.dot_general(
        pooled, fcw_ref[...], (((1,), (0,)), ((), ())),
        preferred_element_type=jnp.float32) + fcb_ref[...]

    li = logits * (1.0 / _TAU)
    li = li - jnp.max(li, axis=-1, keepdims=True)
    e = jnp.exp(li)
    probs = e / jnp.sum(e, axis=-1, keepdims=True)

    pi = probs[:, :, None]
    pj = probs[:, None, :]
    ii = jax.lax.broadcasted_iota(jnp.int32, (_B, 16, 16), 1)
    jj = jax.lax.broadcasted_iota(jnp.int32, (_B, 16, 16), 2)
    g = (pj > pi) | ((pj == pi) & (jj <= ii))
    s = jnp.sum(jnp.where(g, jnp.broadcast_to(pj, (_B, 16, 16)), 0.0), axis=2)
    cnt = jnp.sum(g.astype(jnp.int32), axis=2)
    keep = (s <= _TOP_P) | (cnt < 2)
    masked = jnp.where(keep, probs, 0.0)
    denom = jnp.clip(jnp.sum(masked, axis=-1, keepdims=True), 1e-10, None)
    o_ref[...] = masked / denom


def kernel(patch, conv_w, conv_b, fc_w, fc_b, layer_idx, threshold):
    del layer_idx, threshold
    n_tok = patch.shape[0]
    hw = patch.shape[2] * patch.shape[3]
    p3 = patch.reshape(n_tok, patch.shape[1], hw)

    grid = (n_tok // _B,)
    out = pl.pallas_call(
        _router_body,
        grid=grid,
        in_specs=[
            pl.BlockSpec((_B, p3.shape[1], hw), lambda i: (i, 0, 0)),
            pl.BlockSpec((128, 196), lambda i: (0, 0)),
            pl.BlockSpec((1, 128), lambda i: (0, 0)),
            pl.BlockSpec((128, 16), lambda i: (0, 0)),
            pl.BlockSpec((1, 16), lambda i: (0, 0)),
        ],
        out_specs=pl.BlockSpec((_B, 16), lambda i: (i, 0)),
        out_shape=jax.ShapeDtypeStruct((n_tok, 16), jnp.float32),
        scratch_shapes=[pltpu.VMEM((_B, 128), jnp.float32)],
    )(p3, conv_w, conv_b.reshape(1, 128), fc_w.T, fc_b.reshape(1, 16))
    return out


# speedup vs baseline: 2.8218x; 1.1292x over previous
import jax
import jax.numpy as jnp
from jax.experimental import pallas as pl
from jax.experimental.pallas import tpu as pltpu

_TAU = 0.9
_TOP_P = 0.8
_B = 64


def _router_body(p_ref, we_ref, wo_ref, cb_ref, fcw_ref, fcb_ref, o_ref,
                 pooled_ref):
    we = we_ref[...]
    wo = wo_ref[...]
    cb = cb_ref[...]

    for grp in range(_B // 8):
        toks = [p_ref[grp * 8 + i] for i in range(8)]
        xe = jnp.concatenate([t[:, 0:64] for t in toks], axis=1)
        xo = jnp.concatenate([t[:, 64:128] for t in toks], axis=1)
        h = jax.lax.dot_general(
            xe, we, (((0,), (1,)), ((), ())),
            preferred_element_type=jnp.float32)
        h = h + jax.lax.dot_general(
            xo, wo, (((0,), (1,)), ((), ())),
            preferred_element_type=jnp.float32)
        h = jnp.maximum(h + cb, 0.0)
        hp = jnp.sum(h.reshape(8, 64, 128), axis=1) * (1.0 / 64.0)
        pooled_ref[pl.ds(grp * 8, 8), :] = hp

    pooled = pooled_ref[...]
    logits = jax.lax.dot_general(
        pooled, fcw_ref[...], (((1,), (0,)), ((), ())),
        preferred_element_type=jnp.float32) + fcb_ref[...]

    li = logits * (1.0 / _TAU)
    li = li - jnp.max(li, axis=-1, keepdims=True)
    e = jnp.exp(li)
    probs = e / jnp.sum(e, axis=-1, keepdims=True)

    pi = probs[:, :, None]
    pj = probs[:, None, :]
    ii = jax.lax.broadcasted_iota(jnp.int32, (_B, 16, 16), 1)
    jj = jax.lax.broadcasted_iota(jnp.int32, (_B, 16, 16), 2)
    g = (pj > pi) | ((pj == pi) & (jj <= ii))
    s = jnp.sum(jnp.where(g, jnp.broadcast_to(pj, (_B, 16, 16)), 0.0), axis=2)
    cnt = jnp.sum(g.astype(jnp.int32), axis=2)
    keep = (s <= _TOP_P) | (cnt < 2)
    masked = jnp.where(keep, probs, 0.0)
    denom = jnp.clip(jnp.sum(masked, axis=-1, keepdims=True), 1e-10, None)
    o_ref[...] = masked / denom


def kernel(patch, conv_w, conv_b, fc_w, fc_b, layer_idx, threshold):
    del layer_idx, threshold
    n_tok = patch.shape[0]
    p3 = patch.reshape(n_tok, 98, 128)

    grid = (n_tok // _B,)
    out = pl.pallas_call(
        _router_body,
        grid=grid,
        in_specs=[
            pl.BlockSpec((_B, 98, 128), lambda i: (i, 0, 0)),
            pl.BlockSpec((128, 98), lambda i: (0, 0)),
            pl.BlockSpec((128, 98), lambda i: (0, 0)),
            pl.BlockSpec((1, 128), lambda i: (0, 0)),
            pl.BlockSpec((128, 16), lambda i: (0, 0)),
            pl.BlockSpec((1, 16), lambda i: (0, 0)),
        ],
        out_specs=pl.BlockSpec((_B, 16), lambda i: (i, 0)),
        out_shape=jax.ShapeDtypeStruct((n_tok, 16), jnp.float32),
        scratch_shapes=[pltpu.VMEM((_B, 128), jnp.float32)],
    )(p3, conv_w[:, 0::2], conv_w[:, 1::2], conv_b.reshape(1, 128),
      fc_w.T, fc_b.reshape(1, 16))
    return out
